# baseline (speedup 1.0000x reference)
import jax
import jax.numpy as jnp
from jax.experimental import pallas as pl
from jax.experimental.pallas import tpu as pltpu

_TN = 256
_LC = 128


def _vq_block(xf_ref, x2_ref, cbd_ref, cb_ref, c2f_ref,
              oh_ref, xhat_ref, idx_ref):
    G = cb_ref.shape[0]
    CB = cb_ref.shape[1]
    dim = cb_ref.shape[2]
    TN = xf_ref.shape[0]
    prod = jax.lax.dot_general(
        xf_ref[:, :], cbd_ref[:, :], (((1,), (0,)), ((), ())),
        preferred_element_type=jnp.float32)
    iota_c = jax.lax.broadcasted_iota(jnp.int32, (TN, _LC), 1)
    iota_f = jax.lax.broadcasted_iota(jnp.int32, (TN, CB), 1)
    for g in range(G):
        base = g * CB
        x2g = x2_ref[:, g:g + 1]
        rv = (x2g + c2f_ref[0:1, base:base + _LC]) + prod[:, base:base + _LC]
        ri = iota_c
        for c in range(1, CB // _LC):
            lo = base + c * _LC
            d = (x2g + c2f_ref[0:1, lo:lo + _LC]) + prod[:, lo:lo + _LC]
            upd = d < rv
            ri = jnp.where(upd, iota_c + c * _LC, ri)
            rv = jnp.where(upd, d, rv)
        m = jnp.min(rv, axis=1, keepdims=True)
        cand = jnp.where(rv == m, ri, CB)
        idx = jnp.min(cand, axis=1, keepdims=True)
        oh = (iota_f == idx).astype(jnp.float32)
        oh_ref[:, base:base + CB] = oh
        xhat_ref[:, g * dim:(g + 1) * dim] = jnp.dot(
            oh, cb_ref[g], preferred_element_type=jnp.float32)
        idx_ref[:, g:g + 1] = idx


def kernel(x, code_book):
    n, G, dim = x.shape
    CB = code_book.shape[1]
    xf = x.reshape(n, G * dim)
    x2 = jnp.sum(x * x, axis=-1)
    c2f = jnp.sum(code_book * code_book, axis=-1).reshape(1, G * CB)
    cbm2 = -2.0 * code_book
    cbd = jnp.zeros((G * dim, G * CB), dtype=jnp.float32)
    for g in range(G):
        cbd = cbd.at[g * dim:(g + 1) * dim, g * CB:(g + 1) * CB].set(cbm2[g].T)
    one_hot, x_hat, index = pl.pallas_call(
        _vq_block,
        grid=(n // _TN,),
        in_specs=[
            pl.BlockSpec((_TN, G * dim), lambda i: (i, 0)),
            pl.BlockSpec((_TN, G), lambda i: (i, 0)),
            pl.BlockSpec((G * dim, G * CB), lambda i: (0, 0)),
            pl.BlockSpec((G, CB, dim), lambda i: (0, 0, 0)),
            pl.BlockSpec((1, G * CB), lambda i: (0, 0)),
        ],
        out_specs=[
            pl.BlockSpec((_TN, G * CB), lambda i: (i, 0)),
            pl.BlockSpec((_TN, G * dim), lambda i: (i, 0)),
            pl.BlockSpec((_TN, G), lambda i: (i, 0)),
        ],
        out_shape=[
            jax.ShapeDtypeStruct((n, G * CB), jnp.float32),
            jax.ShapeDtypeStruct((n, G * dim), jnp.float32),
            jax.ShapeDtypeStruct((n, G), jnp.int32),
        ],
        compiler_params=pltpu.CompilerParams(
            dimension_semantics=("parallel",)),
    )(xf, x2, cbd, code_book, c2f)
    return (x_hat.reshape(n, G, dim),
            one_hot.reshape(n, G, CB),
            index[..., None])
